# baseline (speedup 1.0000x reference)
import functools

import jax
import jax.numpy as jnp
from jax import lax
from jax.experimental import pallas as pl
from jax.experimental.pallas import tpu as pltpu
from jax.experimental.pallas import tpu_sc as plsc

EMB = 32
BATCH = 16384
NROW = 1_000_000
NC = 2
NS = 16
NW = NC * NS
BPW = BATCH // NW
G = 16
NGRP = BPW // G
NBLK = G // 8
NPAR = 4


def _make_sc_kernel():
    mesh = plsc.VectorSubcoreMesh(core_axis_name="c", subcore_axis_name="s")

    @functools.partial(
        pl.kernel,
        mesh=mesh,
        out_type=jax.ShapeDtypeStruct((BATCH,), jnp.float32),
        compiler_params=pltpu.CompilerParams(needs_layout_passes=False),
        scratch_types=[
            pltpu.VMEM((BPW,), jnp.int32),
            pltpu.VMEM((BPW,), jnp.int32),
            pltpu.VMEM((BPW,), jnp.int32),
            pltpu.VMEM((NPAR, NBLK, 4, 8, 128), jnp.float32),
            pltpu.VMEM((NPAR, NBLK, 4, 8, 128), jnp.float32),
            pltpu.VMEM((NPAR, NBLK, 4, 8, 128), jnp.float32),
            pltpu.VMEM((BPW,), jnp.float32),
            pltpu.SMEM((3 * G,), jnp.int32),
            [pltpu.SemaphoreType.DMA] * NPAR,
        ],
    )
    def k(ps, qs, rs, P, Q, R, dz, out, pi, qi, ri, pb, qb, rb, ov, so,
          sems):
        wid = lax.axis_index("s") * NC + lax.axis_index("c")

        pltpu.sync_copy(ps.at[wid], pi)
        pltpu.sync_copy(qs.at[wid], qi)
        pltpu.sync_copy(rs.at[wid], ri)

        lane = lax.iota(jnp.int32, 16)
        d0 = lane // 8
        base3 = (lane % 8) * 16

        def fire(g, par, sem):
            for t, iv in enumerate((pi, qi, ri)):
                tv = iv[pl.ds(g * G, 16)]
                gal = (tv // 16) * 16
                for l in range(16):
                    so[t * G + l] = gal[l]

            def one(l, carry):
                blk = l // 8
                lo = pl.multiple_of((l % 8) * 16, 16)
                for t, tbl, buf in ((0, P, pb), (1, Q, qb), (2, R, rb)):
                    off = pl.multiple_of(so[t * G + l], 16)
                    pltpu.async_copy(
                        tbl.at[:, :, pl.ds(off, 16)],
                        buf.at[par, blk, :, :, pl.ds(lo, 16)],
                        sem)
                return carry

            lax.fori_loop(0, G, one, 0)

        def drain(par, sem):
            for buf in (pb, qb, rb):
                pltpu.make_async_copy(dz, buf.at[par], sem).wait()

        def compute(g, par):
            pv = pi[pl.ds(g * G, 16)]
            qv = qi[pl.ds(g * G, 16)]
            rv = ri[pl.ds(g * G, 16)]
            d3p = base3 + (pv & 15)
            d3q = base3 + (qv & 15)
            d3r = base3 + (rv & 15)
            pbp, qbp, rbp = pb.at[par], qb.at[par], rb.at[par]
            acc = jnp.full((16,), 0.0, jnp.float32)
            for j in range(EMB):
                d1 = jnp.full((16,), j // 8, jnp.int32)
                d2 = jnp.full((16,), j % 8, jnp.int32)
                acc = acc + (plsc.load_gather(pbp, [d0, d1, d2, d3p])
                             * plsc.load_gather(qbp, [d0, d1, d2, d3q])
                             * plsc.load_gather(rbp, [d0, d1, d2, d3r]))
            ov[pl.ds(g * G, 16)] = acc

        for s in range(NPAR - 1):
            fire(s, s, sems[s])

        def quad(ii, carry):
            g0 = NPAR * ii
            for s in range(NPAR):
                g = g0 + s

                @pl.when(g + NPAR - 1 < NGRP)
                def _(s=s, g=g):
                    fire(g + NPAR - 1, (s + NPAR - 1) % NPAR,
                         sems[(s + NPAR - 1) % NPAR])

                drain(s, sems[s])
                compute(g, s)
            return carry

        lax.fori_loop(0, NGRP // NPAR, quad, 0)

        pltpu.sync_copy(ov, out.at[pl.ds(wid * BPW, BPW)])

    return k


_sc_kernel = _make_sc_kernel()


def kernel(ps, qs, rs, P, Q, R):
    ps2 = ps.astype(jnp.int32).reshape(NW, BPW)
    qs2 = qs.astype(jnp.int32).reshape(NW, BPW)
    rs2 = rs.astype(jnp.int32).reshape(NW, BPW)
    Pt3 = P.T.reshape(4, 8, NROW)
    Qt3 = Q.T.reshape(4, 8, NROW)
    Rt3 = R.T.reshape(4, 8, NROW)
    dz = jnp.zeros((NBLK, 4, 8, 128), jnp.float32)
    out = _sc_kernel(ps2, qs2, rs2, Pt3, Qt3, Rt3, dz)
    return out.reshape(BATCH, 1)


# speedup vs baseline: 1.1115x; 1.1115x over previous
import functools

import jax
import jax.numpy as jnp
from jax import lax
from jax.experimental import pallas as pl
from jax.experimental.pallas import tpu as pltpu
from jax.experimental.pallas import tpu_sc as plsc

EMB = 32
BATCH = 16384
NROW = 1_000_000
NC = 2
NS = 16
NW = NC * NS
BPW = BATCH // NW
G = 32
NGRP = BPW // G
NBLK = G // 8


def _make_sc_kernel():
    mesh = plsc.VectorSubcoreMesh(core_axis_name="c", subcore_axis_name="s")

    @functools.partial(
        pl.kernel,
        mesh=mesh,
        out_type=jax.ShapeDtypeStruct((BATCH,), jnp.float32),
        compiler_params=pltpu.CompilerParams(needs_layout_passes=False),
        scratch_types=[
            pltpu.VMEM((BPW,), jnp.int32),
            pltpu.VMEM((BPW,), jnp.int32),
            pltpu.VMEM((BPW,), jnp.int32),
            pltpu.VMEM((2, NBLK, 4, 8, 128), jnp.float32),
            pltpu.VMEM((2, NBLK, 4, 8, 128), jnp.float32),
            pltpu.VMEM((2, NBLK, 4, 8, 128), jnp.float32),
            pltpu.VMEM((BPW,), jnp.float32),
            pltpu.SMEM((3 * G,), jnp.int32),
            pltpu.SemaphoreType.DMA,
            pltpu.SemaphoreType.DMA,
        ],
    )
    def k(ps, qs, rs, P, Q, R, dz, out, pi, qi, ri, pb, qb, rb, ov, so,
          sem0, sem1):
        wid = lax.axis_index("s") * NC + lax.axis_index("c")
        base = wid * BPW

        pltpu.sync_copy(ps.at[pl.ds(base, BPW)], pi)
        pltpu.sync_copy(qs.at[pl.ds(base, BPW)], qi)
        pltpu.sync_copy(rs.at[pl.ds(base, BPW)], ri)

        lane = lax.iota(jnp.int32, 16)
        d0 = lane // 8
        base3 = (lane % 8) * 16

        def fire(g, par, sem):
            for t, iv in enumerate((pi, qi, ri)):
                for h in range(G // 16):
                    tv = iv[pl.ds(g * G + h * 16, 16)]
                    gal = (tv // 16) * 16
                    for l in range(16):
                        so[t * G + h * 16 + l] = gal[l]

            def one(l, carry):
                blk = l // 8
                lo = pl.multiple_of((l % 8) * 16, 16)
                for t, tbl, buf in ((0, P, pb), (1, Q, qb), (2, R, rb)):
                    off = pl.multiple_of(so[t * G + l], 16)
                    pltpu.async_copy(
                        tbl.at[:, :, pl.ds(off, 16)],
                        buf.at[par, blk, :, :, pl.ds(lo, 16)],
                        sem)
                return carry

            lax.fori_loop(0, G, one, 0)

        def drain(par, sem):
            for buf in (pb, qb, rb):
                pltpu.make_async_copy(dz, buf.at[par], sem).wait()

        def compute(g, par):
            pbp, qbp, rbp = pb.at[par], qb.at[par], rb.at[par]
            for h in range(G // 16):
                pv = pi[pl.ds(g * G + h * 16, 16)]
                qv = qi[pl.ds(g * G + h * 16, 16)]
                rv = ri[pl.ds(g * G + h * 16, 16)]
                d0h = d0 + 2 * h
                d3p = base3 + (pv & 15)
                d3q = base3 + (qv & 15)
                d3r = base3 + (rv & 15)
                acc = jnp.full((16,), 0.0, jnp.float32)
                for j in range(EMB):
                    d1 = jnp.full((16,), j // 8, jnp.int32)
                    d2 = jnp.full((16,), j % 8, jnp.int32)
                    acc = acc + (plsc.load_gather(pbp, [d0h, d1, d2, d3p])
                                 * plsc.load_gather(qbp, [d0h, d1, d2, d3q])
                                 * plsc.load_gather(rbp, [d0h, d1, d2, d3r]))
                ov[pl.ds(g * G + h * 16, 16)] = acc

        fire(0, 0, sem0)

        def pair(gg, carry):
            g0 = 2 * gg
            fire(g0 + 1, 1, sem1)
            drain(0, sem0)
            compute(g0, 0)

            @pl.when(gg < NGRP // 2 - 1)
            def _():
                fire(g0 + 2, 0, sem0)

            drain(1, sem1)
            compute(g0 + 1, 1)
            return carry

        lax.fori_loop(0, NGRP // 2, pair, 0)

        pltpu.sync_copy(ov, out.at[pl.ds(base, BPW)])

    return k


_sc_kernel = _make_sc_kernel()


def kernel(ps, qs, rs, P, Q, R):
    Pt3 = P.T.reshape(4, 8, NROW)
    Qt3 = Q.T.reshape(4, 8, NROW)
    Rt3 = R.T.reshape(4, 8, NROW)
    dz = jnp.zeros((NBLK, 4, 8, 128), jnp.float32)
    out = _sc_kernel(ps.astype(jnp.int32), qs.astype(jnp.int32),
                     rs.astype(jnp.int32), Pt3, Qt3, Rt3, dz)
    return out.reshape(BATCH, 1)


# speedup vs baseline: 1.1272x; 1.0141x over previous
import functools

import jax
import jax.numpy as jnp
from jax import lax
from jax.experimental import pallas as pl
from jax.experimental.pallas import tpu as pltpu
from jax.experimental.pallas import tpu_sc as plsc

EMB = 32
BATCH = 16384
NROW = 1_000_000
NC = 2
NS = 16
NW = NC * NS
BPW = BATCH // NW
G = 32
NGRP = BPW // G
NBLK = G // 8


def _make_sc_kernel():
    mesh = plsc.VectorSubcoreMesh(core_axis_name="c", subcore_axis_name="s")

    @functools.partial(
        pl.kernel,
        mesh=mesh,
        out_type=jax.ShapeDtypeStruct((BATCH,), jnp.float32),
        compiler_params=pltpu.CompilerParams(needs_layout_passes=False),
        scratch_types=[
            pltpu.VMEM((BPW,), jnp.int32),
            pltpu.VMEM((BPW,), jnp.int32),
            pltpu.VMEM((BPW,), jnp.int32),
            pltpu.VMEM((2, NBLK, 4, 8, 128), jnp.float32),
            pltpu.VMEM((2, NBLK, 4, 8, 128), jnp.float32),
            pltpu.VMEM((2, NBLK, 4, 8, 128), jnp.float32),
            pltpu.VMEM((BPW,), jnp.float32),
            pltpu.SMEM((3 * G,), jnp.int32),
            pltpu.SemaphoreType.DMA,
            pltpu.SemaphoreType.DMA,
        ],
    )
    def k(ps, qs, rs, P, Q, R, dz, out, pi, qi, ri, pb, qb, rb, ov, so,
          sem0, sem1):
        wid = lax.axis_index("s") * NC + lax.axis_index("c")
        base = wid * BPW

        c1 = pltpu.async_copy(ps.at[pl.ds(base, BPW)], pi, sem0)
        c2 = pltpu.async_copy(qs.at[pl.ds(base, BPW)], qi, sem0)
        c3 = pltpu.async_copy(rs.at[pl.ds(base, BPW)], ri, sem0)
        c1.wait()
        c2.wait()
        c3.wait()

        lane = lax.iota(jnp.int32, 16)
        d0 = lane // 8
        base3 = (lane % 8) * 16

        def fire(g, par, sem):
            for t, iv in enumerate((pi, qi, ri)):
                for h in range(G // 16):
                    tv = iv[pl.ds(g * G + h * 16, 16)]
                    gal = (tv // 16) * 16
                    for l in range(16):
                        so[t * G + h * 16 + l] = gal[l]

            def one(l, carry):
                blk = l // 8
                lo = pl.multiple_of((l % 8) * 16, 16)
                for t, tbl, buf in ((0, P, pb), (1, Q, qb), (2, R, rb)):
                    off = pl.multiple_of(so[t * G + l], 16)
                    pltpu.async_copy(
                        tbl.at[:, :, pl.ds(off, 16)],
                        buf.at[par, blk, :, :, pl.ds(lo, 16)],
                        sem)
                return carry

            lax.fori_loop(0, G, one, 0)

        def drain(par, sem):
            for buf in (pb, qb, rb):
                pltpu.make_async_copy(dz, buf.at[par], sem).wait()

        def compute(g, par):
            pbp, qbp, rbp = pb.at[par], qb.at[par], rb.at[par]
            for h in range(G // 16):
                pv = pi[pl.ds(g * G + h * 16, 16)]
                qv = qi[pl.ds(g * G + h * 16, 16)]
                rv = ri[pl.ds(g * G + h * 16, 16)]
                d0h = d0 + 2 * h
                d3p = base3 + (pv & 15)
                d3q = base3 + (qv & 15)
                d3r = base3 + (rv & 15)
                acc = jnp.full((16,), 0.0, jnp.float32)
                for j in range(EMB):
                    d1 = jnp.full((16,), j // 8, jnp.int32)
                    d2 = jnp.full((16,), j % 8, jnp.int32)
                    acc = acc + (plsc.load_gather(pbp, [d0h, d1, d2, d3p])
                                 * plsc.load_gather(qbp, [d0h, d1, d2, d3q])
                                 * plsc.load_gather(rbp, [d0h, d1, d2, d3r]))
                ov[pl.ds(g * G + h * 16, 16)] = acc

        fire(0, 0, sem0)

        def pair(gg, carry):
            g0 = 2 * gg
            fire(g0 + 1, 1, sem1)
            drain(0, sem0)
            compute(g0, 0)

            @pl.when(gg < NGRP // 2 - 1)
            def _():
                fire(g0 + 2, 0, sem0)

            drain(1, sem1)
            compute(g0 + 1, 1)
            return carry

        lax.fori_loop(0, NGRP // 2, pair, 0)

        pltpu.sync_copy(ov, out.at[pl.ds(base, BPW)])

    return k


_sc_kernel = _make_sc_kernel()


def kernel(ps, qs, rs, P, Q, R):
    Pt3 = P.T.reshape(4, 8, NROW)
    Qt3 = Q.T.reshape(4, 8, NROW)
    Rt3 = R.T.reshape(4, 8, NROW)
    dz = jnp.zeros((NBLK, 4, 8, 128), jnp.float32)
    out = _sc_kernel(ps.astype(jnp.int32), qs.astype(jnp.int32),
                     rs.astype(jnp.int32), Pt3, Qt3, Rt3, dz)
    return out.reshape(BATCH, 1)
